# baseline (speedup 1.0000x reference)
import functools

import jax
import jax.numpy as jnp
from jax import lax
from jax.experimental import pallas as pl
from jax.experimental.pallas import tpu as pltpu
from jax.experimental.pallas import tpu_sc as plsc

_EMBED = 64
_NC = 2
_NS = 16
_NW = _NC * _NS
_NR = 16


@functools.lru_cache(maxsize=None)
def _make_sc_gather(bsz, fields):
    r_per_w = bsz // _NW
    SG = r_per_w // _NR
    assert r_per_w % _NR == 0 and SG % 2 == 0 and SG >= 4
    mesh = plsc.VectorSubcoreMesh(core_axis_name="c", subcore_axis_name="s")

    @functools.partial(
        pl.kernel,
        out_type=jax.ShapeDtypeStruct((bsz, fields, _EMBED), jnp.float32),
        mesh=mesh,
        scratch_types=[
            pltpu.VMEM((r_per_w, fields), jnp.int32),
            pltpu.VMEM((2, _NR, fields, _EMBED), jnp.float32),
            pltpu.SemaphoreType.DMA,
            pltpu.SemaphoreType.DMA,
            pltpu.SemaphoreType.DMA,
            pltpu.SemaphoreType.DMA,
        ],
        compiler_params=pltpu.CompilerParams(use_tc_tiling_on_sc=False),
    )
    def k(table_hbm, idx_hbm, out_hbm, idx_v, rows_v, g0, g1, w0, w1):
        wid = lax.axis_index("s") * _NC + lax.axis_index("c")
        rbase = wid * r_per_w
        pltpu.sync_copy(idx_hbm.at[pl.ds(rbase, r_per_w)], idx_v)
        gsem = (g0, g1)
        wsem = (w0, w1)

        def gather_desc(s, bank, r):
            return pltpu.make_async_copy(
                table_hbm.at[idx_v.at[s * _NR + r]], rows_v.at[bank, r],
                gsem[bank])

        def write_desc(s, bank):
            return pltpu.make_async_copy(
                rows_v.at[bank],
                out_hbm.at[pl.ds(rbase + s * _NR, _NR)], wsem[bank])

        def fire_gathers(s, bank):
            for r in range(_NR):
                gather_desc(s, bank, r).start()

        def drain_gathers(s, bank):
            for r in range(_NR):
                gather_desc(s, bank, r).wait()

        def fire_writes(s, bank):
            write_desc(s, bank).start()

        def drain_writes(s, bank):
            write_desc(s, bank).wait()

        def step(s, bank):
            drain_gathers(s, bank)
            drain_writes(s - 1, 1 - bank)
            fire_gathers(s + 1, 1 - bank)
            fire_writes(s, bank)

        fire_gathers(0, 0)
        drain_gathers(0, 0)
        fire_gathers(1, 1)
        fire_writes(0, 0)

        def body(t, carry):
            step(2 * t + 1, 1)
            step(2 * t + 2, 0)
            return carry

        lax.fori_loop(0, (SG - 2) // 2, body, 0)

        drain_gathers(SG - 1, 1)
        drain_writes(SG - 2, 0)
        fire_writes(SG - 1, 1)
        drain_writes(SG - 1, 1)

    return k


def kernel(idx, weight):
    bsz, fields = idx.shape
    return _make_sc_gather(bsz, fields)(weight, jnp.asarray(idx, jnp.int32))


# speedup vs baseline: 1.2166x; 1.2166x over previous
import functools

import jax
import jax.numpy as jnp
from jax import lax
from jax.experimental import pallas as pl
from jax.experimental.pallas import tpu as pltpu
from jax.experimental.pallas import tpu_sc as plsc

_EMBED = 64
_NC = 2
_NS = 16
_NW = _NC * _NS
_NR = 16


@functools.lru_cache(maxsize=None)
def _make_sc_gather(bsz, fields):
    r_per_w = bsz // _NW
    SG = r_per_w // _NR
    assert r_per_w % _NR == 0 and SG % 2 == 0 and SG >= 4
    mesh = plsc.VectorSubcoreMesh(core_axis_name="c", subcore_axis_name="s")

    @functools.partial(
        pl.kernel,
        out_type=jax.ShapeDtypeStruct((bsz, fields, _EMBED), jnp.float32),
        mesh=mesh,
        scratch_types=[
            pltpu.VMEM((r_per_w, fields), jnp.int32),
            pltpu.VMEM((2, _NR, fields, _EMBED), jnp.float32),
            pltpu.SemaphoreType.DMA,
            pltpu.SemaphoreType.DMA,
            pltpu.SemaphoreType.DMA,
            pltpu.SemaphoreType.DMA,
        ],
        compiler_params=pltpu.CompilerParams(use_tc_tiling_on_sc=False),
    )
    def k(table_hbm, idx_hbm, out_hbm, idx_v, rows_v, g0, g1, w0, w1):
        wid = lax.axis_index("s") * _NC + lax.axis_index("c")
        rbase = wid * r_per_w
        pltpu.sync_copy(idx_hbm.at[pl.ds(rbase, r_per_w)], idx_v)
        gsem = (g0, g1)
        wsem = (w0, w1)

        def gather_desc(s, bank, r):
            return pltpu.make_async_copy(
                table_hbm.at[idx_v.at[s * _NR + r]], rows_v.at[bank, r],
                gsem[bank])

        def write_desc(s, bank):
            return pltpu.make_async_copy(
                rows_v.at[bank],
                out_hbm.at[pl.ds(rbase + s * _NR, _NR)], wsem[bank])

        def fire_gathers(s, bank):
            for r in range(_NR):
                gather_desc(s, bank, r).start()

        def drain_gathers(s, bank):
            for r in range(_NR):
                gather_desc(s, bank, r).wait()

        def fire_writes(s, bank):
            write_desc(s, bank).start()

        def drain_writes(s, bank):
            write_desc(s, bank).wait()

        def step(s, bank):
            drain_gathers(s, bank)
            drain_writes(s - 1, 1 - bank)
            fire_gathers(s + 1, 1 - bank)
            fire_writes(s, bank)

        fire_gathers(0, 0)
        drain_gathers(0, 0)
        fire_gathers(1, 1)
        fire_writes(0, 0)

        def body(t, carry):
            step(2 * t + 1, 1)
            step(2 * t + 2, 0)
            return carry

        lax.fori_loop(0, (SG - 2) // 2, body, 0)

        drain_gathers(SG - 1, 1)
        drain_writes(SG - 2, 0)
        fire_writes(SG - 1, 1)
        drain_writes(SG - 1, 1)

    return k


def _tc_transpose_block(a_ref, o_ref, nsub):
    x = a_ref[...]
    j = x.shape[0] // nsub
    x = x.reshape(j, nsub, 128)
    for u in range(nsub):
        o_ref[u * 128:(u + 1) * 128, :] = x[:, u, :].T


@functools.lru_cache(maxsize=None)
def _make_tc_transpose(bsz, fields):
    row = fields * _EMBED
    nsub = row // 128
    J = 512
    assert row % 128 == 0 and bsz % J == 0
    grid = bsz // J
    return pl.pallas_call(
        functools.partial(_tc_transpose_block, nsub=nsub),
        grid=(grid,),
        in_specs=[pl.BlockSpec((J * nsub, 128), lambda i: (i, 0))],
        out_specs=pl.BlockSpec((row, J), lambda i: (0, i)),
        out_shape=jax.ShapeDtypeStruct((row, bsz), jnp.float32),
    )


def kernel(idx, weight):
    bsz, fields = idx.shape
    flat = _make_sc_gather(bsz, fields)(weight, jnp.asarray(idx, jnp.int32))
    a = flat.reshape(bsz * fields * _EMBED // 128, 128)
    outt = _make_tc_transpose(bsz, fields)(a)
    return jnp.transpose(outt.reshape(fields, _EMBED, bsz), (2, 0, 1))


# speedup vs baseline: 1.2238x; 1.0059x over previous
import functools

import jax
import jax.numpy as jnp
from jax import lax
from jax.experimental import pallas as pl
from jax.experimental.pallas import tpu as pltpu
from jax.experimental.pallas import tpu_sc as plsc

_EMBED = 64
_NC = 2
_NS = 16
_NW = _NC * _NS
_NR = 16


@functools.lru_cache(maxsize=None)
def _make_sc_gather(bsz, fields):
    r_per_w = bsz // _NW
    SG = r_per_w // _NR
    assert r_per_w % _NR == 0 and SG % 2 == 0 and SG >= 4
    mesh = plsc.VectorSubcoreMesh(core_axis_name="c", subcore_axis_name="s")

    @functools.partial(
        pl.kernel,
        out_type=jax.ShapeDtypeStruct((bsz, fields, _EMBED), jnp.float32),
        mesh=mesh,
        scratch_types=[
            pltpu.VMEM((r_per_w, fields), jnp.int32),
            pltpu.VMEM((2, _NR, fields, _EMBED), jnp.float32),
            pltpu.SemaphoreType.DMA,
            pltpu.SemaphoreType.DMA,
            pltpu.SemaphoreType.DMA,
            pltpu.SemaphoreType.DMA,
        ],
        compiler_params=pltpu.CompilerParams(use_tc_tiling_on_sc=False),
    )
    def k(table_hbm, idx_hbm, out_hbm, idx_v, rows_v, g0, g1, w0, w1):
        wid = lax.axis_index("s") * _NC + lax.axis_index("c")
        rbase = wid * r_per_w
        pltpu.sync_copy(idx_hbm.at[pl.ds(rbase, r_per_w)], idx_v)
        gsem = (g0, g1)
        wsem = (w0, w1)

        def gather_desc(s, bank, r):
            return pltpu.make_async_copy(
                table_hbm.at[idx_v.at[s * _NR + r]], rows_v.at[bank, r],
                gsem[bank])

        def write_desc(s, bank):
            return pltpu.make_async_copy(
                rows_v.at[bank],
                out_hbm.at[pl.ds(rbase + s * _NR, _NR)], wsem[bank])

        def fire_gathers(s, bank):
            for r in range(_NR):
                gather_desc(s, bank, r).start()

        def drain_gathers(s, bank):
            for r in range(_NR):
                gather_desc(s, bank, r).wait()

        def fire_writes(s, bank):
            write_desc(s, bank).start()

        def drain_writes(s, bank):
            write_desc(s, bank).wait()

        def step(s, bank):
            drain_gathers(s, bank)
            drain_writes(s - 1, 1 - bank)
            fire_gathers(s + 1, 1 - bank)
            fire_writes(s, bank)

        fire_gathers(0, 0)
        drain_gathers(0, 0)
        fire_gathers(1, 1)
        fire_writes(0, 0)

        def body(t, carry):
            step(2 * t + 1, 1)
            step(2 * t + 2, 0)
            return carry

        lax.fori_loop(0, (SG - 2) // 2, body, 0)

        drain_gathers(SG - 1, 1)
        drain_writes(SG - 2, 0)
        fire_writes(SG - 1, 1)
        drain_writes(SG - 1, 1)

    return k


def _tc_transpose_block(a_ref, o_ref, nsub):
    x = a_ref[...]
    j = x.shape[0] // nsub
    x = x.reshape(j, nsub, 128)
    for u in range(nsub):
        o_ref[u * 128:(u + 1) * 128, :] = x[:, u, :].T


@functools.lru_cache(maxsize=None)
def _make_tc_transpose(bsz, fields):
    row = fields * _EMBED
    nsub = row // 128
    J = 1024
    assert row % 128 == 0 and bsz % J == 0
    grid = bsz // J
    return pl.pallas_call(
        functools.partial(_tc_transpose_block, nsub=nsub),
        grid=(grid,),
        in_specs=[pl.BlockSpec((J * nsub, 128), lambda i: (i, 0))],
        out_specs=pl.BlockSpec((row, J), lambda i: (0, i)),
        out_shape=jax.ShapeDtypeStruct((row, bsz), jnp.float32),
    )


def kernel(idx, weight):
    bsz, fields = idx.shape
    flat = _make_sc_gather(bsz, fields)(weight, jnp.asarray(idx, jnp.int32))
    a = flat.reshape(bsz * fields * _EMBED // 128, 128)
    outt = _make_tc_transpose(bsz, fields)(a)
    return jnp.transpose(outt.reshape(fields, _EMBED, bsz), (2, 0, 1))
